# baseline (device time: 153107 ns/iter reference)
import jax
import jax.numpy as jnp
from jax import lax
from jax.experimental import pallas as pl
from jax.experimental.pallas import tpu as pltpu

N_DEV = 32


def kernel(x, w_mat):
    m, k = x.shape
    _, n = w_mat.shape
    m_per = m // N_DEV

    def body(x_ref, w_ref, out_ref, wb_ref, send_ref, recv_ref,
             send_sems, recv_sems):
        d = lax.axis_index("i")
        left = lax.rem(d + N_DEV - 1, N_DEV)
        right = lax.rem(d + 1, N_DEV)

        barrier_sem = pltpu.get_barrier_semaphore()
        pl.semaphore_signal(barrier_sem, inc=1, device_id=(left,),
                            device_id_type=pl.DeviceIdType.MESH)
        pl.semaphore_signal(barrier_sem, inc=1, device_id=(right,),
                            device_id_type=pl.DeviceIdType.MESH)
        pl.semaphore_wait(barrier_sem, 2)

        wb_ref[...] = w_ref[...].astype(jnp.bfloat16)

        def partial_chunk(c):
            xc = x_ref[pl.ds(c * m_per, m_per), :].astype(jnp.bfloat16)
            return jnp.dot(xc, wb_ref[...], preferred_element_type=jnp.float32)

        send_ref[0] = partial_chunk(lax.rem(d + N_DEV - 1, N_DEV)).astype(
            jnp.bfloat16)

        for s in range(N_DEV - 1):
            slot = s % 2
            rdma = pltpu.make_async_remote_copy(
                src_ref=send_ref.at[slot],
                dst_ref=recv_ref.at[s],
                send_sem=send_sems.at[slot],
                recv_sem=recv_sems.at[s],
                device_id=(right,),
                device_id_type=pl.DeviceIdType.MESH,
            )
            rdma.start()
            c_r = lax.rem(d + 2 * N_DEV - 2 - s, N_DEV)
            add_term = partial_chunk(c_r)
            rdma.wait()
            acc = recv_ref[s].astype(jnp.float32) + add_term
            if s < N_DEV - 2:
                send_ref[(s + 1) % 2] = acc.astype(jnp.bfloat16)
            else:
                out_ref[...] = acc

    return pl.pallas_call(
        body,
        out_shape=jax.ShapeDtypeStruct((m_per, n), jnp.float32),
        in_specs=[pl.BlockSpec(memory_space=pltpu.VMEM)] * 2,
        out_specs=pl.BlockSpec(memory_space=pltpu.VMEM),
        scratch_shapes=[
            pltpu.VMEM((k, n), jnp.bfloat16),
            pltpu.VMEM((2, m_per, n), jnp.bfloat16),
            pltpu.VMEM((N_DEV - 1, m_per, n), jnp.bfloat16),
            pltpu.SemaphoreType.DMA((2,)),
            pltpu.SemaphoreType.DMA((N_DEV - 1,)),
        ],
        compiler_params=pltpu.CompilerParams(collective_id=0),
    )(x, w_mat)
